# Initial kernel scaffold; baseline (speedup 1.0000x reference)
import jax
import jax.numpy as jnp
from jax.experimental import pallas as pl


def kernel(x, emb):
    raise NotImplementedError("write your pallas kernel here")


# speedup vs baseline: 1.4422x; 1.4422x over previous
import jax
import jax.numpy as jnp
from jax.experimental import pallas as pl
from jax.experimental.pallas import tpu as pltpu

_WD = 32
_NE = 64
_BLK = 3584
_COST = 1.0 + 0.25


def _vq_body(x_ref, emb_ref, out_ref, loss_ref):
    j = pl.program_id(1)
    xb = x_ref[0]
    emb = emb_ref[...]
    e_sq = jnp.sum(emb * emb, axis=1)[:, None]
    x_sq = jnp.sum(xb * xb, axis=0, keepdims=True)
    prod = jax.lax.dot_general(
        emb, xb, (((1,), (0,)), ((), ())),
        preferred_element_type=jnp.float32,
        precision=jax.lax.Precision.DEFAULT)
    d = (x_sq + e_sq) - 2.0 * prod
    m = jnp.min(d, axis=0, keepdims=True)
    iota = jax.lax.broadcasted_iota(jnp.int32, (_NE, _BLK), 0)
    sel = jnp.min(jnp.where(d == m, iota, _NE), axis=0, keepdims=True)
    onehot = (iota == sel).astype(jnp.float32)
    q = jax.lax.dot_general(
        emb, onehot, (((0,), (0,)), ((), ())),
        preferred_element_type=jnp.float32,
        precision=jax.lax.Precision.HIGHEST)
    out_ref[0] = q

    part = jnp.sum(m)

    @pl.when(j == 0)
    def _init():
        loss_ref[...] = jnp.zeros_like(loss_ref)

    loss_ref[...] += jnp.full(loss_ref.shape, part, jnp.float32)


def kernel(x, emb):
    b, c, h, w = x.shape
    n = h * w
    xr = x.reshape(b, c, n)
    grid = (b, n // _BLK)
    out3, partials = pl.pallas_call(
        _vq_body,
        grid=grid,
        in_specs=[
            pl.BlockSpec((1, c, _BLK), lambda i, j: (i, 0, j)),
            pl.BlockSpec((_NE, _WD), lambda i, j: (0, 0)),
        ],
        out_specs=[
            pl.BlockSpec((1, c, _BLK), lambda i, j: (i, 0, j)),
            pl.BlockSpec((1, 1, 128), lambda i, j: (i, 0, 0)),
        ],
        out_shape=[
            jax.ShapeDtypeStruct((b, c, n), jnp.float32),
            jax.ShapeDtypeStruct((b, 1, 128), jnp.float32),
        ],
        compiler_params=pltpu.CompilerParams(
            dimension_semantics=("parallel", "arbitrary")),
    )(xr, emb)
    out = out3.reshape(b, c, h, w)
    loss = (jnp.sum(partials[:, 0, 0]) * (_COST / x.size)).astype(jnp.float32)
    return (loss, out, 1, emb)


# speedup vs baseline: 1.5654x; 1.0854x over previous
import jax
import jax.numpy as jnp
import jax.scipy.linalg
from jax.experimental import pallas as pl
from jax.experimental.pallas import tpu as pltpu

_WD = 32
_NE = 64
_PK = 4
_BLK = 3584
_W = _BLK // _PK
_COST = 1.0 + 0.25


def _vq_body(x_ref, embd_ref, embtd_ref, out_ref, loss_ref):
    j = pl.program_id(1)
    embd = embd_ref[...]
    embtd = embtd_ref[...]
    e_sq = jnp.sum(embd * embd, axis=1, keepdims=True)
    xb = x_ref[0]
    xs = jnp.concatenate(
        [xb[:, k * _W:(k + 1) * _W] for k in range(_PK)], axis=0)
    xsq_g = jnp.sum((xs * xs).reshape(_PK, _WD, _W), axis=1)
    xsq4 = jnp.reshape(
        jnp.broadcast_to(xsq_g[:, None, :], (_PK, _NE, _W)), (_PK * _NE, _W))
    prod = jax.lax.dot_general(
        embd, xs, (((1,), (0,)), ((), ())),
        preferred_element_type=jnp.float32,
        precision=jax.lax.Precision.DEFAULT)
    d = ((xsq4 + e_sq) - 2.0 * prod).reshape(_PK, _NE, _W)
    m = jnp.min(d, axis=1, keepdims=True)
    iota = jax.lax.broadcasted_iota(jnp.int32, (_PK, _NE, _W), 1)
    sel = jnp.min(jnp.where(d == m, iota, _NE), axis=1, keepdims=True)
    onehot = (iota == sel).astype(jnp.float32).reshape(_PK * _NE, _W)
    q4 = jax.lax.dot_general(
        embtd, onehot, (((1,), (0,)), ((), ())),
        preferred_element_type=jnp.float32,
        precision=jax.lax.Precision.DEFAULT)
    out_ref[0] = jnp.concatenate(
        [q4[k * _WD:(k + 1) * _WD, :] for k in range(_PK)], axis=1)

    part = jnp.sum(m)

    @pl.when(j == 0)
    def _init():
        loss_ref[...] = jnp.zeros_like(loss_ref)

    loss_ref[...] += jnp.full(loss_ref.shape, part, jnp.float32)


def kernel(x, emb):
    b, c, h, w = x.shape
    n = h * w
    xr = x.reshape(b, c, n)
    embd = jax.scipy.linalg.block_diag(*([emb] * _PK))
    embtd = jax.scipy.linalg.block_diag(*([emb.T] * _PK))
    grid = (b, n // _BLK)
    out3, partials = pl.pallas_call(
        _vq_body,
        grid=grid,
        in_specs=[
            pl.BlockSpec((1, c, _BLK), lambda i, j: (i, 0, j)),
            pl.BlockSpec((_PK * _NE, _PK * _WD), lambda i, j: (0, 0)),
            pl.BlockSpec((_PK * _WD, _PK * _NE), lambda i, j: (0, 0)),
        ],
        out_specs=[
            pl.BlockSpec((1, c, _BLK), lambda i, j: (i, 0, j)),
            pl.BlockSpec((1, 1, 128), lambda i, j: (i, 0, 0)),
        ],
        out_shape=[
            jax.ShapeDtypeStruct((b, c, n), jnp.float32),
            jax.ShapeDtypeStruct((b, 1, 128), jnp.float32),
        ],
        compiler_params=pltpu.CompilerParams(
            dimension_semantics=("parallel", "arbitrary")),
    )(xr, embd, embtd)
    out = out3.reshape(b, c, h, w)
    loss = (jnp.sum(partials[:, 0, 0]) * (_COST / x.size)).astype(jnp.float32)
    return (loss, out, 1, emb)


# speedup vs baseline: 1.7277x; 1.1037x over previous
import jax
import jax.numpy as jnp
from jax.experimental import pallas as pl
from jax.experimental.pallas import tpu as pltpu

_WD = 32
_NE = 64
_HC = 8
_PK = 4
_COST = 1.0 + 0.25


def _vq_body(x_ref, lA_ref, lB_ref, l2A_ref, l2B_ref, out_ref, loss_ref):
    j = pl.program_id(1)
    lA = lA_ref[...]
    lB = lB_ref[...]
    e_sq = jnp.sum(lA * lA, axis=1, keepdims=True)
    xb = x_ref[0]
    xs = xb.reshape(_WD * _HC, 224)
    xsq8 = jnp.sum(xb * xb, axis=0)

    w = 224
    halves = []
    for g, l_ref in ((0, lA), (1, lB)):
        prod = jax.lax.dot_general(
            l_ref, xs, (((1,), (0,)), ((), ())),
            preferred_element_type=jnp.float32,
            precision=jax.lax.Precision.DEFAULT)
        xsqh = jnp.reshape(
            jnp.broadcast_to(xsq8[g * _PK:(g + 1) * _PK][:, None, :],
                             (_PK, _NE, w)), (_PK * _NE, w))
        d = ((xsqh + e_sq) - 2.0 * prod).reshape(_PK, _NE, w)
        m = jnp.min(d, axis=1, keepdims=True)
        iota = jax.lax.broadcasted_iota(jnp.int32, (_PK, _NE, w), 1)
        sel = jnp.min(jnp.where(d == m, iota, _NE), axis=1, keepdims=True)
        onehot = (iota == sel).astype(jnp.float32).reshape(_PK * _NE, w)
        halves.append((onehot, jnp.sum(m)))

    q = (jax.lax.dot_general(
            l2A_ref[...], halves[0][0], (((1,), (0,)), ((), ())),
            preferred_element_type=jnp.float32,
            precision=jax.lax.Precision.DEFAULT)
         + jax.lax.dot_general(
            l2B_ref[...], halves[1][0], (((1,), (0,)), ((), ())),
            preferred_element_type=jnp.float32,
            precision=jax.lax.Precision.DEFAULT))
    out_ref[0] = q.reshape(_WD, _HC, 224)

    part = halves[0][1] + halves[1][1]

    @pl.when(j == 0)
    def _init():
        loss_ref[...] = jnp.zeros_like(loss_ref)

    loss_ref[...] += jnp.full(loss_ref.shape, part, jnp.float32)


def _mk_lhs(emb, h_offsets):
    blocks = []
    for h in h_offsets:
        t = jnp.zeros((_NE, _WD, _HC), emb.dtype).at[:, :, h].set(emb)
        blocks.append(t.reshape(_NE, _WD * _HC))
    return jnp.concatenate(blocks, axis=0)


def kernel(x, emb):
    b, c, h, w = x.shape
    lA = _mk_lhs(emb, [0, 1, 2, 3])
    lB = _mk_lhs(emb, [4, 5, 6, 7])
    l2A, l2B = lA.T, lB.T
    grid = (b, h // _HC)
    out, partials = pl.pallas_call(
        _vq_body,
        grid=grid,
        in_specs=[
            pl.BlockSpec((1, c, _HC, w), lambda i, j: (i, 0, j, 0)),
            pl.BlockSpec((256, 256), lambda i, j: (0, 0)),
            pl.BlockSpec((256, 256), lambda i, j: (0, 0)),
            pl.BlockSpec((256, 256), lambda i, j: (0, 0)),
            pl.BlockSpec((256, 256), lambda i, j: (0, 0)),
        ],
        out_specs=[
            pl.BlockSpec((1, c, _HC, w), lambda i, j: (i, 0, j, 0)),
            pl.BlockSpec((1, 1, 128), lambda i, j: (i, 0, 0)),
        ],
        out_shape=[
            jax.ShapeDtypeStruct((b, c, h, w), jnp.float32),
            jax.ShapeDtypeStruct((b, 1, 128), jnp.float32),
        ],
        compiler_params=pltpu.CompilerParams(
            dimension_semantics=("parallel", "arbitrary")),
    )(x, lA, lB, l2A, l2B)
    loss = (jnp.sum(partials[:, 0, 0]) * (_COST / x.size)).astype(jnp.float32)
    return (loss, out, 1, emb)


# speedup vs baseline: 2.6538x; 1.5361x over previous
import jax
import jax.numpy as jnp
from jax.experimental import pallas as pl
from jax.experimental.pallas import tpu as pltpu

_WD = 32
_NE = 64
_HC = 16
_PK = 4
_COST = 1.0 + 0.25


def _vq_body(x_ref, lA_ref, lB_ref, l2A_ref, l2B_ref, esq_ref,
             out_ref, loss_ref):
    j = pl.program_id(1)
    lA = lA_ref[...]
    lB = lB_ref[...]
    e_sq = esq_ref[...][:, 0:1]
    xb = x_ref[0]
    w = 224

    parts = []
    outs = []
    for g in range(_HC // 8):
        xg = xb[:, g * 8:(g + 1) * 8, :]
        xs = xg.reshape(_WD * 8, w)
        xsq8 = jnp.sum(xg * xg, axis=0)
        ohs = []
        for half, l in ((0, lA), (1, lB)):
            prod = jax.lax.dot_general(
                l, xs, (((1,), (0,)), ((), ())),
                preferred_element_type=jnp.float32,
                precision=jax.lax.Precision.DEFAULT)
            xsqh = jnp.reshape(
                jnp.broadcast_to(xsq8[half * _PK:(half + 1) * _PK][:, None, :],
                                 (_PK, _NE, w)), (_PK * _NE, w))
            d = ((xsqh + e_sq) + prod).reshape(_PK, _NE, w)
            m = jnp.min(d, axis=1, keepdims=True)
            iota = jax.lax.broadcasted_iota(jnp.int32, (_PK, _NE, w), 1)
            sel = jnp.min(jnp.where(d == m, iota, _NE), axis=1, keepdims=True)
            ohs.append((iota == sel).astype(jnp.float32).reshape(_PK * _NE, w))
            parts.append(jnp.sum(m))
        q = (jax.lax.dot_general(
                l2A_ref[...], ohs[0], (((1,), (0,)), ((), ())),
                preferred_element_type=jnp.float32,
                precision=jax.lax.Precision.DEFAULT)
             + jax.lax.dot_general(
                l2B_ref[...], ohs[1], (((1,), (0,)), ((), ())),
                preferred_element_type=jnp.float32,
                precision=jax.lax.Precision.DEFAULT))
        outs.append(q.reshape(_WD, 8, w))

    out_ref[0] = jnp.concatenate(outs, axis=1)
    part = parts[0] + parts[1] + parts[2] + parts[3]

    @pl.when(j == 0)
    def _init():
        loss_ref[...] = jnp.zeros_like(loss_ref)

    loss_ref[...] += jnp.full(loss_ref.shape, part, jnp.float32)


def _mk_lhs(emb, h_offsets, scale):
    blocks = []
    for h in h_offsets:
        t = jnp.zeros((_NE, _WD, 8), emb.dtype).at[:, :, h].set(scale * emb)
        blocks.append(t.reshape(_NE, _WD * 8))
    return jnp.concatenate(blocks, axis=0)


def kernel(x, emb):
    b, c, h, w = x.shape
    lA = _mk_lhs(emb, [0, 1, 2, 3], -2.0)
    lB = _mk_lhs(emb, [4, 5, 6, 7], -2.0)
    l2A = _mk_lhs(emb, [0, 1, 2, 3], 1.0).T
    l2B = _mk_lhs(emb, [4, 5, 6, 7], 1.0).T
    e_sq = jnp.broadcast_to(
        jnp.tile(jnp.sum(emb * emb, axis=1), 4)[:, None], (256, 128))
    grid = (b, h // _HC)
    out, partials = pl.pallas_call(
        _vq_body,
        grid=grid,
        in_specs=[
            pl.BlockSpec((1, c, _HC, w), lambda i, j: (i, 0, j, 0)),
            pl.BlockSpec((256, 256), lambda i, j: (0, 0)),
            pl.BlockSpec((256, 256), lambda i, j: (0, 0)),
            pl.BlockSpec((256, 256), lambda i, j: (0, 0)),
            pl.BlockSpec((256, 256), lambda i, j: (0, 0)),
            pl.BlockSpec((256, 128), lambda i, j: (0, 0)),
        ],
        out_specs=[
            pl.BlockSpec((1, c, _HC, w), lambda i, j: (i, 0, j, 0)),
            pl.BlockSpec((1, 1, 128), lambda i, j: (i, 0, 0)),
        ],
        out_shape=[
            jax.ShapeDtypeStruct((b, c, h, w), jnp.float32),
            jax.ShapeDtypeStruct((b, 1, 128), jnp.float32),
        ],
        compiler_params=pltpu.CompilerParams(
            dimension_semantics=("parallel", "arbitrary")),
    )(x, lA, lB, l2A, l2B, e_sq)
    loss = (jnp.sum(partials[:, 0, 0]) * (_COST / x.size)).astype(jnp.float32)
    return (loss, out, 1, emb)


# speedup vs baseline: 4.2055x; 1.5847x over previous
import jax
import jax.numpy as jnp
from jax.experimental import pallas as pl
from jax.experimental.pallas import tpu as pltpu

_WD = 32
_NE = 64
_HC = 56
_PK = 4
_COST = 1.0 + 0.25


def _vq_body(x_ref, lA_ref, lB_ref, l2A_ref, l2B_ref, esq_ref,
             out_ref, loss_ref):
    j = pl.program_id(1)
    lA = lA_ref[...]
    lB = lB_ref[...]
    e_sq = esq_ref[...][:, 0:1]
    xb = x_ref[0]
    w = 224

    parts = []
    outs = []
    for g in range(_HC // 8):
        xg = xb[:, g * 8:(g + 1) * 8, :]
        xs = xg.reshape(_WD * 8, w)
        xsq8 = jnp.sum(xg * xg, axis=0)
        ohs = []
        for half, l in ((0, lA), (1, lB)):
            prod = jax.lax.dot_general(
                l, xs, (((1,), (0,)), ((), ())),
                preferred_element_type=jnp.float32,
                precision=jax.lax.Precision.DEFAULT)
            xsqh = jnp.reshape(
                jnp.broadcast_to(xsq8[half * _PK:(half + 1) * _PK][:, None, :],
                                 (_PK, _NE, w)), (_PK * _NE, w))
            d = ((xsqh + e_sq) + prod).reshape(_PK, _NE, w)
            m = jnp.min(d, axis=1, keepdims=True)
            iota = jax.lax.broadcasted_iota(jnp.int32, (_PK, _NE, w), 1)
            sel = jnp.min(jnp.where(d == m, iota, _NE), axis=1, keepdims=True)
            ohs.append((iota == sel).astype(jnp.float32).reshape(_PK * _NE, w))
            parts.append(jnp.sum(m))
        q = (jax.lax.dot_general(
                l2A_ref[...], ohs[0], (((1,), (0,)), ((), ())),
                preferred_element_type=jnp.float32,
                precision=jax.lax.Precision.DEFAULT)
             + jax.lax.dot_general(
                l2B_ref[...], ohs[1], (((1,), (0,)), ((), ())),
                preferred_element_type=jnp.float32,
                precision=jax.lax.Precision.DEFAULT))
        outs.append(q.reshape(_WD, 8, w))

    out_ref[0] = jnp.concatenate(outs, axis=1)
    part = sum(parts)

    @pl.when(j == 0)
    def _init():
        loss_ref[...] = jnp.zeros_like(loss_ref)

    loss_ref[...] += jnp.full(loss_ref.shape, part, jnp.float32)


def _mk_lhs(emb, h_offsets, scale):
    blocks = []
    for h in h_offsets:
        t = jnp.zeros((_NE, _WD, 8), emb.dtype).at[:, :, h].set(scale * emb)
        blocks.append(t.reshape(_NE, _WD * 8))
    return jnp.concatenate(blocks, axis=0)


def kernel(x, emb):
    b, c, h, w = x.shape
    lA = _mk_lhs(emb, [0, 1, 2, 3], -2.0)
    lB = _mk_lhs(emb, [4, 5, 6, 7], -2.0)
    l2A = _mk_lhs(emb, [0, 1, 2, 3], 1.0).T
    l2B = _mk_lhs(emb, [4, 5, 6, 7], 1.0).T
    e_sq = jnp.broadcast_to(
        jnp.tile(jnp.sum(emb * emb, axis=1), 4)[:, None], (256, 128))
    grid = (b, h // _HC)
    out, partials = pl.pallas_call(
        _vq_body,
        grid=grid,
        in_specs=[
            pl.BlockSpec((1, c, _HC, w), lambda i, j: (i, 0, j, 0)),
            pl.BlockSpec((256, 256), lambda i, j: (0, 0)),
            pl.BlockSpec((256, 256), lambda i, j: (0, 0)),
            pl.BlockSpec((256, 256), lambda i, j: (0, 0)),
            pl.BlockSpec((256, 256), lambda i, j: (0, 0)),
            pl.BlockSpec((256, 128), lambda i, j: (0, 0)),
        ],
        out_specs=[
            pl.BlockSpec((1, c, _HC, w), lambda i, j: (i, 0, j, 0)),
            pl.BlockSpec((1, 1, 128), lambda i, j: (i, 0, 0)),
        ],
        out_shape=[
            jax.ShapeDtypeStruct((b, c, h, w), jnp.float32),
            jax.ShapeDtypeStruct((b, 1, 128), jnp.float32),
        ],
        compiler_params=pltpu.CompilerParams(
            dimension_semantics=("parallel", "arbitrary")),
    )(x, lA, lB, l2A, l2B, e_sq)
    loss = (jnp.sum(partials[:, 0, 0]) * (_COST / x.size)).astype(jnp.float32)
    return (loss, out, 1, emb)


# speedup vs baseline: 4.5663x; 1.0858x over previous
import jax
import jax.numpy as jnp
from jax.experimental import pallas as pl
from jax.experimental.pallas import tpu as pltpu

_WD = 32
_NE = 64
_HC = 112
_PK = 4
_COST = 1.0 + 0.25


def _vq_body(x_ref, lA_ref, lB_ref, l2A_ref, l2B_ref, esq_ref,
             out_ref, loss_ref):
    j = pl.program_id(1)
    lA = lA_ref[...]
    lB = lB_ref[...]
    e_sq = esq_ref[...][:, 0:1]
    xb = x_ref[0]
    w = 224

    parts = []
    outs = []
    for g in range(_HC // 8):
        xg = xb[:, g * 8:(g + 1) * 8, :]
        xs = xg.reshape(_WD * 8, w)
        xsq8 = jnp.sum(xg * xg, axis=0)
        ohs = []
        for half, l in ((0, lA), (1, lB)):
            prod = jax.lax.dot_general(
                l, xs, (((1,), (0,)), ((), ())),
                preferred_element_type=jnp.float32,
                precision=jax.lax.Precision.DEFAULT)
            xsqh = jnp.reshape(
                jnp.broadcast_to(xsq8[half * _PK:(half + 1) * _PK][:, None, :],
                                 (_PK, _NE, w)), (_PK * _NE, w))
            d = ((xsqh + e_sq) + prod).reshape(_PK, _NE, w)
            m = jnp.min(d, axis=1, keepdims=True)
            iota = jax.lax.broadcasted_iota(jnp.int32, (_PK, _NE, w), 1)
            sel = jnp.min(jnp.where(d == m, iota, _NE), axis=1, keepdims=True)
            ohs.append((iota == sel).astype(jnp.float32).reshape(_PK * _NE, w))
            parts.append(jnp.sum(m))
        q = (jax.lax.dot_general(
                l2A_ref[...], ohs[0], (((1,), (0,)), ((), ())),
                preferred_element_type=jnp.float32,
                precision=jax.lax.Precision.DEFAULT)
             + jax.lax.dot_general(
                l2B_ref[...], ohs[1], (((1,), (0,)), ((), ())),
                preferred_element_type=jnp.float32,
                precision=jax.lax.Precision.DEFAULT))
        outs.append(q.reshape(_WD, 8, w))

    out_ref[0] = jnp.concatenate(outs, axis=1)
    part = sum(parts)

    @pl.when(j == 0)
    def _init():
        loss_ref[...] = jnp.zeros_like(loss_ref)

    loss_ref[...] += jnp.full(loss_ref.shape, part, jnp.float32)


def _mk_lhs(emb, h_offsets, scale):
    blocks = []
    for h in h_offsets:
        t = jnp.zeros((_NE, _WD, 8), emb.dtype).at[:, :, h].set(scale * emb)
        blocks.append(t.reshape(_NE, _WD * 8))
    return jnp.concatenate(blocks, axis=0)


def kernel(x, emb):
    b, c, h, w = x.shape
    lA = _mk_lhs(emb, [0, 1, 2, 3], -2.0)
    lB = _mk_lhs(emb, [4, 5, 6, 7], -2.0)
    l2A = _mk_lhs(emb, [0, 1, 2, 3], 1.0).T
    l2B = _mk_lhs(emb, [4, 5, 6, 7], 1.0).T
    e_sq = jnp.broadcast_to(
        jnp.tile(jnp.sum(emb * emb, axis=1), 4)[:, None], (256, 128))
    grid = (b, h // _HC)
    out, partials = pl.pallas_call(
        _vq_body,
        grid=grid,
        in_specs=[
            pl.BlockSpec((1, c, _HC, w), lambda i, j: (i, 0, j, 0)),
            pl.BlockSpec((256, 256), lambda i, j: (0, 0)),
            pl.BlockSpec((256, 256), lambda i, j: (0, 0)),
            pl.BlockSpec((256, 256), lambda i, j: (0, 0)),
            pl.BlockSpec((256, 256), lambda i, j: (0, 0)),
            pl.BlockSpec((256, 128), lambda i, j: (0, 0)),
        ],
        out_specs=[
            pl.BlockSpec((1, c, _HC, w), lambda i, j: (i, 0, j, 0)),
            pl.BlockSpec((1, 1, 128), lambda i, j: (i, 0, 0)),
        ],
        out_shape=[
            jax.ShapeDtypeStruct((b, c, h, w), jnp.float32),
            jax.ShapeDtypeStruct((b, 1, 128), jnp.float32),
        ],
        compiler_params=pltpu.CompilerParams(
            dimension_semantics=("parallel", "arbitrary")),
    )(x, lA, lB, l2A, l2B, e_sq)
    loss = (jnp.sum(partials[:, 0, 0]) * (_COST / x.size)).astype(jnp.float32)
    return (loss, out, 1, emb)


# speedup vs baseline: 5.2330x; 1.1460x over previous
import jax
import jax.numpy as jnp
from jax.experimental import pallas as pl
from jax.experimental.pallas import tpu as pltpu

_WD = 32
_NE = 64
_HC = 112
_PK = 4
_COST = 1.0 + 0.25


def _vq_body(x_ref, lA_ref, lB_ref, l2A_ref, l2B_ref, esq_ref,
             out_ref, loss_ref):
    j = pl.program_id(1)
    lA = lA_ref[...]
    lB = lB_ref[...]
    e_sq = esq_ref[...][:, 0:1]
    xb = x_ref[0]
    w = 224

    parts = []
    outs = []
    for g in range(_HC // 8):
        xg = xb[:, g * 8:(g + 1) * 8, :]
        xs = xg.reshape(_WD * 8, w)
        xsq8 = jnp.sum(xg * xg, axis=0)
        ohs = []
        for half, l in ((0, lA), (1, lB)):
            prod = jax.lax.dot_general(
                l, xs, (((1,), (0,)), ((), ())),
                preferred_element_type=jnp.float32,
                precision=jax.lax.Precision.DEFAULT)
            xsqh = xsq8[half * _PK:(half + 1) * _PK][:, None, :]
            d = ((xsqh + e_sq.reshape(_PK, _NE, 1))
                 + prod.reshape(_PK, _NE, w))
            iota = jax.lax.broadcasted_iota(jnp.int32, (_PK, _NE, w), 1)
            dv, iv = d, iota
            for width in (_NE, _NE // 2, _NE // 4):
                half_w = width // 2
                av, bv = dv[:, :half_w], dv[:, half_w:width]
                ai, bi = iv[:, :half_w], iv[:, half_w:width]
                take_b = bv < av
                dv = jnp.where(take_b, bv, av)
                iv = jnp.where(take_b, bi, ai)
            m = jnp.min(dv, axis=1, keepdims=True)
            sel = jnp.min(jnp.where(dv == m, iv, _NE), axis=1, keepdims=True)
            ohs.append((iota == sel).astype(jnp.float32).reshape(_PK * _NE, w))
            parts.append(jnp.sum(m))
        q = (jax.lax.dot_general(
                l2A_ref[...], ohs[0], (((1,), (0,)), ((), ())),
                preferred_element_type=jnp.float32,
                precision=jax.lax.Precision.DEFAULT)
             + jax.lax.dot_general(
                l2B_ref[...], ohs[1], (((1,), (0,)), ((), ())),
                preferred_element_type=jnp.float32,
                precision=jax.lax.Precision.DEFAULT))
        outs.append(q.reshape(_WD, 8, w))

    out_ref[0] = jnp.concatenate(outs, axis=1)
    part = sum(parts)

    @pl.when(j == 0)
    def _init():
        loss_ref[...] = jnp.zeros_like(loss_ref)

    loss_ref[...] += jnp.full(loss_ref.shape, part, jnp.float32)


def _mk_lhs(emb, h_offsets, scale):
    blocks = []
    for h in h_offsets:
        t = jnp.zeros((_NE, _WD, 8), emb.dtype).at[:, :, h].set(scale * emb)
        blocks.append(t.reshape(_NE, _WD * 8))
    return jnp.concatenate(blocks, axis=0)


def kernel(x, emb):
    b, c, h, w = x.shape
    lA = _mk_lhs(emb, [0, 1, 2, 3], -2.0)
    lB = _mk_lhs(emb, [4, 5, 6, 7], -2.0)
    l2A = _mk_lhs(emb, [0, 1, 2, 3], 1.0).T
    l2B = _mk_lhs(emb, [4, 5, 6, 7], 1.0).T
    e_sq = jnp.broadcast_to(
        jnp.tile(jnp.sum(emb * emb, axis=1), 4)[:, None], (256, 128))
    grid = (b, h // _HC)
    out, partials = pl.pallas_call(
        _vq_body,
        grid=grid,
        in_specs=[
            pl.BlockSpec((1, c, _HC, w), lambda i, j: (i, 0, j, 0)),
            pl.BlockSpec((256, 256), lambda i, j: (0, 0)),
            pl.BlockSpec((256, 256), lambda i, j: (0, 0)),
            pl.BlockSpec((256, 256), lambda i, j: (0, 0)),
            pl.BlockSpec((256, 256), lambda i, j: (0, 0)),
            pl.BlockSpec((256, 128), lambda i, j: (0, 0)),
        ],
        out_specs=[
            pl.BlockSpec((1, c, _HC, w), lambda i, j: (i, 0, j, 0)),
            pl.BlockSpec((1, 1, 128), lambda i, j: (i, 0, 0)),
        ],
        out_shape=[
            jax.ShapeDtypeStruct((b, c, h, w), jnp.float32),
            jax.ShapeDtypeStruct((b, 1, 128), jnp.float32),
        ],
        compiler_params=pltpu.CompilerParams(
            dimension_semantics=("parallel", "arbitrary")),
    )(x, lA, lB, l2A, l2B, e_sq)
    loss = (jnp.sum(partials[:, 0, 0]) * (_COST / x.size)).astype(jnp.float32)
    return (loss, out, 1, emb)
